# baseline (speedup 1.0000x reference)
import jax
import jax.numpy as jnp
from jax.experimental import pallas as pl
from jax.experimental.pallas import tpu as pltpu

_SUB = 8


def _row_body(z_ref, w_ref, b_ref, o_ref):
    o_ref[...] = w_ref[...] + b_ref[...]


def kernel(z, W, b):
    batch, seq = z.shape
    n = batch * seq
    num_states, vocab = W.shape
    assert vocab % _SUB == 0
    vs = vocab // _SUB

    zf = z.reshape(n).astype(jnp.int32)
    W3 = W.reshape(num_states, _SUB, vs)
    b3 = b.reshape(1, _SUB, vs)

    out = pl.pallas_call(
        _row_body,
        grid_spec=pltpu.PrefetchScalarGridSpec(
            num_scalar_prefetch=1,
            grid=(n,),
            in_specs=[
                pl.BlockSpec((1, _SUB, vs), lambda i, zr: (zr[i], 0, 0)),
                pl.BlockSpec((1, _SUB, vs), lambda i, zr: (0, 0, 0)),
            ],
            out_specs=pl.BlockSpec((1, _SUB, vs), lambda i, zr: (i, 0, 0)),
        ),
        out_shape=jax.ShapeDtypeStruct((n, _SUB, vs), jnp.float32),
    )(zf, W3, b3)
    return out.reshape(batch, seq, vocab)


# speedup vs baseline: 1.2133x; 1.2133x over previous
import jax
import jax.numpy as jnp
from jax.experimental import pallas as pl
from jax.experimental.pallas import tpu as pltpu

_SUB = 8
_K = 16


def _body(z_ref, w_hbm, b_ref, o_hbm, in_bufs, out_bufs, in_sems, out_sems):
    n = o_hbm.shape[0]

    def in_copy(slot, row):
        return pltpu.make_async_copy(
            w_hbm.at[pl.ds(z_ref[row], 1)], in_bufs.at[pl.ds(slot, 1)],
            in_sems.at[slot])

    def out_copy(slot, row):
        return pltpu.make_async_copy(
            out_bufs.at[pl.ds(slot, 1)], o_hbm.at[pl.ds(row, 1)],
            out_sems.at[slot])

    def warmup(k, _):
        in_copy(k, k).start()
        return _

    jax.lax.fori_loop(0, _K, warmup, None)

    def step(i, _):
        j = jax.lax.rem(i, _K)
        in_copy(j, i).wait()

        @pl.when(i >= _K)
        def _free_out_slot():
            out_copy(j, i - _K).wait()

        out_bufs[pl.ds(j, 1)] = in_bufs[pl.ds(j, 1)] + b_ref[...]
        out_copy(j, i).start()

        @pl.when(i + _K < n)
        def _refill():
            in_copy(j, i + _K).start()

        return _

    jax.lax.fori_loop(0, n, step, None)

    def final_wait(i, _):
        out_copy(jax.lax.rem(i, _K), i).wait()
        return _

    jax.lax.fori_loop(n - _K, n, final_wait, None)


def kernel(z, W, b):
    batch, seq = z.shape
    n = batch * seq
    num_states, vocab = W.shape
    assert vocab % _SUB == 0
    vs = vocab // _SUB

    zf = z.reshape(n).astype(jnp.int32)
    W3 = W.reshape(num_states, _SUB, vs)
    b3 = b.reshape(1, _SUB, vs)

    out = pl.pallas_call(
        _body,
        grid_spec=pltpu.PrefetchScalarGridSpec(
            num_scalar_prefetch=1,
            grid=(1,),
            in_specs=[
                pl.BlockSpec(memory_space=pltpu.MemorySpace.HBM),
                pl.BlockSpec((1, _SUB, vs), lambda i, zr: (0, 0, 0)),
            ],
            out_specs=pl.BlockSpec(memory_space=pltpu.MemorySpace.HBM),
            scratch_shapes=[
                pltpu.VMEM((_K, _SUB, vs), jnp.float32),
                pltpu.VMEM((_K, _SUB, vs), jnp.float32),
                pltpu.SemaphoreType.DMA((_K,)),
                pltpu.SemaphoreType.DMA((_K,)),
            ],
        ),
        out_shape=jax.ShapeDtypeStruct((n, _SUB, vs), jnp.float32),
    )(zf, W3, b3)
    return out.reshape(batch, seq, vocab)
